# Initial kernel scaffold; baseline (speedup 1.0000x reference)
import functools

import jax
import jax.numpy as jnp
from jax import lax
from jax.experimental import pallas as pl
from jax.experimental.pallas import tpu as pltpu
from jax.experimental.pallas import tpu_sc as plsc

N = 10000
E = 320000
D = 128

NC = 2
NS = 16
NW = NC * NS
EPT = E // NW
CHUNK = 80
NCHUNK = EPT // CHUNK
ROWS_PER_TILE = N // NS
ZR = 125


def _mp_kernel(h_hbm, src_hbm, dst_hbm, out_hbm, sidx, didx, rows, acc, sem):
    c = lax.axis_index("c")
    s = lax.axis_index("s")
    wid = c * NS + s

    def zero_full(i, carry):
        for j in range(D // 16):
            rows[i, pl.ds(j * 16, 16)] = jnp.zeros((16,), jnp.float32)
        return carry
    lax.fori_loop(0, CHUNK, zero_full, 0, unroll=4)

    r0 = s * ROWS_PER_TILE
    nfull = ROWS_PER_TILE // CHUNK
    for k in range(nfull):
        pltpu.sync_copy(rows, acc.at[pl.ds(r0 + k * CHUNK, CHUNK), :])
    rem = ROWS_PER_TILE - nfull * CHUNK
    pltpu.sync_copy(rows.at[pl.ds(0, rem), :],
                    acc.at[pl.ds(r0 + nfull * CHUNK, rem), :])

    plsc.subcore_barrier()

    ebase = wid * EPT

    def step(k, carry):
        base = ebase + k * CHUNK
        pltpu.sync_copy(src_hbm.at[pl.ds(base, CHUNK)], sidx)
        pltpu.sync_copy(dst_hbm.at[pl.ds(base, CHUNK)], didx)
        pltpu.async_copy(h_hbm.at[sidx], rows, sem).wait()
        pltpu.sync_copy(rows, acc.at[didx], add=True)
        return carry

    lax.fori_loop(0, NCHUNK, step, 0)

    plsc.subcore_barrier()

    for k in range(ROWS_PER_TILE // ZR):
        pltpu.sync_copy(acc.at[pl.ds(r0 + k * ZR, ZR), :],
                        out_hbm.at[c, pl.ds(r0 + k * ZR, ZR), :])


def _message_pass(h, src, dst):
    mesh = plsc.VectorSubcoreMesh(core_axis_name="c", subcore_axis_name="s",
                                  num_cores=NC, num_subcores=NS)
    return pl.kernel(
        _mp_kernel,
        out_type=jax.ShapeDtypeStruct((NC, N, D), jnp.float32),
        mesh=mesh,
        scratch_types=[
            pltpu.VMEM((CHUNK,), jnp.int32),
            pltpu.VMEM((CHUNK,), jnp.int32),
            pltpu.VMEM((CHUNK, D), jnp.float32),
            pltpu.VMEM_SHARED((N, D), jnp.float32),
            pltpu.SemaphoreType.DMA,
        ],
    )(h, src, dst)


ROWB = 1000


def _mm_kernel(x_ref, w_ref, o_ref):
    o_ref[...] = jnp.dot(x_ref[...], w_ref[...],
                         preferred_element_type=jnp.float32)


def _matmul(x, w):
    return pl.pallas_call(
        _mm_kernel,
        grid=(N // ROWB,),
        in_specs=[
            pl.BlockSpec((ROWB, D), lambda i: (i, 0)),
            pl.BlockSpec((D, D), lambda i: (0, 0)),
        ],
        out_specs=pl.BlockSpec((ROWB, D), lambda i: (i, 0)),
        out_shape=jax.ShapeDtypeStruct((N, D), jnp.float32),
    )(x, w)


def _fuse_kernel(a_ref, b_ref, w_ref, o_ref):
    z = a_ref[0] + a_ref[1] + b_ref[...]
    z = jnp.maximum(z, 0.0)
    o_ref[...] = jnp.dot(z, w_ref[...], preferred_element_type=jnp.float32)


def _relu_matmul(parts, b, w):
    return pl.pallas_call(
        _fuse_kernel,
        grid=(N // ROWB,),
        in_specs=[
            pl.BlockSpec((NC, ROWB, D), lambda i: (0, i, 0)),
            pl.BlockSpec((1, D), lambda i: (0, 0)),
            pl.BlockSpec((D, D), lambda i: (0, 0)),
        ],
        out_specs=pl.BlockSpec((ROWB, D), lambda i: (i, 0)),
        out_shape=jax.ShapeDtypeStruct((N, D), jnp.float32),
    )(parts, b.reshape(1, D), w)


def _lsm_kernel(a_ref, b_ref, o_ref):
    t = a_ref[0] + a_ref[1] + b_ref[...]
    m = jnp.max(t, axis=-1, keepdims=True)
    e = jnp.exp(t - m)
    lse = jnp.log(jnp.sum(e, axis=-1, keepdims=True)) + m
    o_ref[...] = t - lse


def _log_softmax(parts, b):
    return pl.pallas_call(
        _lsm_kernel,
        grid=(N // ROWB,),
        in_specs=[
            pl.BlockSpec((NC, ROWB, D), lambda i: (0, i, 0)),
            pl.BlockSpec((1, D), lambda i: (0, 0)),
        ],
        out_specs=pl.BlockSpec((ROWB, D), lambda i: (i, 0)),
        out_shape=jax.ShapeDtypeStruct((N, D), jnp.float32),
    )(parts, b.reshape(1, D))


def kernel(x, edge_index, W1, b1, W2, b2, W3, b3):
    src = edge_index[0].astype(jnp.int32)
    dst = edge_index[1].astype(jnp.int32)

    h = _matmul(x, W1)
    parts = _message_pass(h, src, dst)
    h = _relu_matmul(parts, b1, W2)
    parts = _message_pass(h, src, dst)
    h = _relu_matmul(parts, b2, W3)
    parts = _message_pass(h, src, dst)
    return _log_softmax(parts, b3)


# speedup vs baseline: 5.0777x; 5.0777x over previous
import functools

import jax
import jax.numpy as jnp
from jax import lax
from jax.experimental import pallas as pl
from jax.experimental.pallas import tpu as pltpu
from jax.experimental.pallas import tpu_sc as plsc

N = 10000
E = 320000
D = 128

NC = 2
NS = 16
NW = NC * NS
EPT = E // NW
CHUNK = 80
NCHUNK = EPT // CHUNK
RCHUNK = 80
NRCHUNK = N // RCHUNK


def _mp_kernel(h_hbm, src_hbm, dst_hbm, out_hbm, sidx, didx, rows, acc, sem):
    c = lax.axis_index("c")
    s = lax.axis_index("s")
    wid = c * NS + s

    def zero_full(i, carry):
        for j in range(D // 16):
            rows[i, pl.ds(j * 16, 16)] = jnp.zeros((16,), jnp.float32)
        return carry
    lax.fori_loop(0, CHUNK, zero_full, 0, unroll=4)

    nmine = (NRCHUNK - s + NS - 1) // NS

    def zero_acc(i, carry):
        r0 = pl.multiple_of((s + i * NS) * RCHUNK, 8)
        pltpu.sync_copy(rows, acc.at[pl.ds(r0, RCHUNK), :])
        return carry

    lax.fori_loop(0, nmine, zero_acc, 0)

    plsc.subcore_barrier()

    ebase = wid * EPT

    def step(k, carry):
        base = ebase + k * CHUNK
        pltpu.sync_copy(src_hbm.at[pl.ds(base, CHUNK)], sidx)
        pltpu.sync_copy(dst_hbm.at[pl.ds(base, CHUNK)], didx)
        pltpu.async_copy(h_hbm.at[sidx], rows, sem).wait()
        pltpu.sync_copy(rows, acc.at[didx], add=True)
        return carry

    lax.fori_loop(0, NCHUNK, step, 0)

    plsc.subcore_barrier()

    def wout(i, carry):
        r0 = pl.multiple_of((s + i * NS) * RCHUNK, 8)
        pltpu.sync_copy(acc.at[pl.ds(r0, RCHUNK), :],
                        out_hbm.at[c, pl.ds(r0, RCHUNK), :])
        return carry

    lax.fori_loop(0, nmine, wout, 0)


def _message_pass(h, src, dst):
    mesh = plsc.VectorSubcoreMesh(core_axis_name="c", subcore_axis_name="s",
                                  num_cores=NC, num_subcores=NS)
    return pl.kernel(
        _mp_kernel,
        out_type=jax.ShapeDtypeStruct((NC, N, D), jnp.float32),
        mesh=mesh,
        scratch_types=[
            pltpu.VMEM((CHUNK,), jnp.int32),
            pltpu.VMEM((CHUNK,), jnp.int32),
            pltpu.VMEM((CHUNK, D), jnp.float32),
            pltpu.VMEM_SHARED((N, D), jnp.float32),
            pltpu.SemaphoreType.DMA,
        ],
    )(h, src, dst)


ROWB = 1000


def _mm_kernel(x_ref, w_ref, o_ref):
    o_ref[...] = jnp.dot(x_ref[...], w_ref[...],
                         preferred_element_type=jnp.float32)


def _matmul(x, w):
    return pl.pallas_call(
        _mm_kernel,
        grid=(N // ROWB,),
        in_specs=[
            pl.BlockSpec((ROWB, D), lambda i: (i, 0)),
            pl.BlockSpec((D, D), lambda i: (0, 0)),
        ],
        out_specs=pl.BlockSpec((ROWB, D), lambda i: (i, 0)),
        out_shape=jax.ShapeDtypeStruct((N, D), jnp.float32),
    )(x, w)


def _fuse_kernel(a_ref, b_ref, w_ref, o_ref):
    z = a_ref[0] + a_ref[1] + b_ref[...]
    z = jnp.maximum(z, 0.0)
    o_ref[...] = jnp.dot(z, w_ref[...], preferred_element_type=jnp.float32)


def _relu_matmul(parts, b, w):
    return pl.pallas_call(
        _fuse_kernel,
        grid=(N // ROWB,),
        in_specs=[
            pl.BlockSpec((NC, ROWB, D), lambda i: (0, i, 0)),
            pl.BlockSpec((1, D), lambda i: (0, 0)),
            pl.BlockSpec((D, D), lambda i: (0, 0)),
        ],
        out_specs=pl.BlockSpec((ROWB, D), lambda i: (i, 0)),
        out_shape=jax.ShapeDtypeStruct((N, D), jnp.float32),
    )(parts, b.reshape(1, D), w)


def _lsm_kernel(a_ref, b_ref, o_ref):
    t = a_ref[0] + a_ref[1] + b_ref[...]
    m = jnp.max(t, axis=-1, keepdims=True)
    e = jnp.exp(t - m)
    lse = jnp.log(jnp.sum(e, axis=-1, keepdims=True)) + m
    o_ref[...] = t - lse


def _log_softmax(parts, b):
    return pl.pallas_call(
        _lsm_kernel,
        grid=(N // ROWB,),
        in_specs=[
            pl.BlockSpec((NC, ROWB, D), lambda i: (0, i, 0)),
            pl.BlockSpec((1, D), lambda i: (0, 0)),
        ],
        out_specs=pl.BlockSpec((ROWB, D), lambda i: (i, 0)),
        out_shape=jax.ShapeDtypeStruct((N, D), jnp.float32),
    )(parts, b.reshape(1, D))


def kernel(x, edge_index, W1, b1, W2, b2, W3, b3):
    src = edge_index[0].astype(jnp.int32)
    dst = edge_index[1].astype(jnp.int32)

    h = _matmul(x, W1)
    parts = _message_pass(h, src, dst)
    h = _relu_matmul(parts, b1, W2)
    parts = _message_pass(h, src, dst)
    h = _relu_matmul(parts, b2, W3)
    parts = _message_pass(h, src, dst)
    return _log_softmax(parts, b3)


# speedup vs baseline: 12.1949x; 2.4017x over previous
import functools

import jax
import jax.numpy as jnp
from jax import lax
from jax.experimental import pallas as pl
from jax.experimental.pallas import tpu as pltpu
from jax.experimental.pallas import tpu_sc as plsc

N = 10000
E = 320000
D = 128

NC = 2
NS = 16
NW = NC * NS
EPT = E // NW
CHUNK = 80
NCHUNK = EPT // CHUNK
RCHUNK = 80
NRCHUNK = N // RCHUNK


def _mp_kernel(h_hbm, src_hbm, dst_hbm, out_hbm,
               sidx, didx, rows_a, rows_b, acc, sem_a, sem_b):
    c = lax.axis_index("c")
    s = lax.axis_index("s")
    wid = c * NS + s

    ld_s = pltpu.async_copy(src_hbm.at[wid], sidx, sem_a)
    ld_d = pltpu.async_copy(dst_hbm.at[wid], didx, sem_b)

    def zero_full(i, carry):
        for j in range(D // 16):
            rows_a[i, pl.ds(j * 16, 16)] = jnp.zeros((16,), jnp.float32)
        return carry
    lax.fori_loop(0, CHUNK, zero_full, 0, unroll=4)

    nmine = (NRCHUNK - s + NS - 1) // NS

    def zero_acc(i, carry):
        r0 = pl.multiple_of((s + i * NS) * RCHUNK, 8)
        pltpu.sync_copy(rows_a, acc.at[pl.ds(r0, RCHUNK), :])
        return carry

    lax.fori_loop(0, nmine, zero_acc, 0)

    ld_s.wait()
    ld_d.wait()
    plsc.subcore_barrier()

    ga = pltpu.async_copy(h_hbm.at[sidx.at[0]], rows_a, sem_a)

    def pipe(j, carry):
        c0 = 2 * j
        gb = pltpu.async_copy(h_hbm.at[sidx.at[c0 + 1]], rows_b, sem_b)
        pltpu.make_async_copy(h_hbm.at[sidx.at[c0]], rows_a, sem_a).wait()
        pltpu.sync_copy(rows_a, acc.at[didx.at[c0]], add=True)
        pltpu.async_copy(h_hbm.at[sidx.at[c0 + 2]], rows_a, sem_a)
        gb.wait()
        pltpu.sync_copy(rows_b, acc.at[didx.at[c0 + 1]], add=True)
        return carry

    lax.fori_loop(0, (NCHUNK - 1) // 2, pipe, 0)
    pltpu.make_async_copy(h_hbm.at[sidx.at[NCHUNK - 1]], rows_a, sem_a).wait()
    pltpu.sync_copy(rows_a, acc.at[didx.at[NCHUNK - 1]], add=True)

    plsc.subcore_barrier()

    def wout(i, carry):
        r0 = pl.multiple_of((s + i * NS) * RCHUNK, 8)
        pltpu.sync_copy(acc.at[pl.ds(r0, RCHUNK), :],
                        out_hbm.at[c, pl.ds(r0, RCHUNK), :])
        return carry

    lax.fori_loop(0, nmine, wout, 0)


def _message_pass(h, src, dst):
    mesh = plsc.VectorSubcoreMesh(core_axis_name="c", subcore_axis_name="s",
                                  num_cores=NC, num_subcores=NS)
    return pl.kernel(
        _mp_kernel,
        out_type=jax.ShapeDtypeStruct((NC, N, D), jnp.float32),
        mesh=mesh,
        compiler_params=pltpu.CompilerParams(use_tc_tiling_on_sc=False),
        scratch_types=[
            pltpu.VMEM((NCHUNK, CHUNK), jnp.int32),
            pltpu.VMEM((NCHUNK, CHUNK), jnp.int32),
            pltpu.VMEM((CHUNK, D), jnp.float32),
            pltpu.VMEM((CHUNK, D), jnp.float32),
            pltpu.VMEM_SHARED((N, D), jnp.float32),
            pltpu.SemaphoreType.DMA,
            pltpu.SemaphoreType.DMA,
        ],
    )(h, src, dst)


ROWB = 1000


def _mm_kernel(x_ref, w_ref, o_ref):
    o_ref[...] = jnp.dot(x_ref[...], w_ref[...],
                         preferred_element_type=jnp.float32)


def _matmul(x, w):
    return pl.pallas_call(
        _mm_kernel,
        grid=(N // ROWB,),
        in_specs=[
            pl.BlockSpec((ROWB, D), lambda i: (i, 0)),
            pl.BlockSpec((D, D), lambda i: (0, 0)),
        ],
        out_specs=pl.BlockSpec((ROWB, D), lambda i: (i, 0)),
        out_shape=jax.ShapeDtypeStruct((N, D), jnp.float32),
    )(x, w)


def _fuse_kernel(a_ref, b_ref, w_ref, o_ref):
    z = a_ref[0] + a_ref[1] + b_ref[...]
    z = jnp.maximum(z, 0.0)
    o_ref[...] = jnp.dot(z, w_ref[...], preferred_element_type=jnp.float32)


def _relu_matmul(parts, b, w):
    return pl.pallas_call(
        _fuse_kernel,
        grid=(N // ROWB,),
        in_specs=[
            pl.BlockSpec((NC, ROWB, D), lambda i: (0, i, 0)),
            pl.BlockSpec((1, D), lambda i: (0, 0)),
            pl.BlockSpec((D, D), lambda i: (0, 0)),
        ],
        out_specs=pl.BlockSpec((ROWB, D), lambda i: (i, 0)),
        out_shape=jax.ShapeDtypeStruct((N, D), jnp.float32),
    )(parts, b.reshape(1, D), w)


def _lsm_kernel(a_ref, b_ref, o_ref):
    t = a_ref[0] + a_ref[1] + b_ref[...]
    m = jnp.max(t, axis=-1, keepdims=True)
    e = jnp.exp(t - m)
    lse = jnp.log(jnp.sum(e, axis=-1, keepdims=True)) + m
    o_ref[...] = t - lse


def _log_softmax(parts, b):
    return pl.pallas_call(
        _lsm_kernel,
        grid=(N // ROWB,),
        in_specs=[
            pl.BlockSpec((NC, ROWB, D), lambda i: (0, i, 0)),
            pl.BlockSpec((1, D), lambda i: (0, 0)),
        ],
        out_specs=pl.BlockSpec((ROWB, D), lambda i: (i, 0)),
        out_shape=jax.ShapeDtypeStruct((N, D), jnp.float32),
    )(parts, b.reshape(1, D))


def kernel(x, edge_index, W1, b1, W2, b2, W3, b3):
    src = edge_index[0].astype(jnp.int32).reshape(NW, NCHUNK, CHUNK)
    dst = edge_index[1].astype(jnp.int32).reshape(NW, NCHUNK, CHUNK)

    h = _matmul(x, W1)
    parts = _message_pass(h, src, dst)
    h = _relu_matmul(parts, b1, W2)
    parts = _message_pass(h, src, dst)
    h = _relu_matmul(parts, b2, W3)
    parts = _message_pass(h, src, dst)
    return _log_softmax(parts, b3)
